# baseline (speedup 1.0000x reference)
import jax
import jax.numpy as jnp
from jax import lax
from jax.experimental import pallas as pl
from jax.experimental.pallas import tpu as pltpu

J = 17
E = 32
F_IN = 3
H = 64
C_OUT = 16
JF = J * F_IN
JFP = JF + 1
JH = J * H
BT = 2048


def _prep(ei_ref, w1_ref, b1_ref, w2_ref, b2_ref,
          g1_s, g2_s, bb2_s):
    f32 = jnp.float32
    i32 = jnp.int32
    ei = ei_ref[...]
    src = ei[0:1, :]
    dst = ei[1:2, :]
    jrow = lax.broadcasted_iota(i32, (J, E), 0)
    S = (jrow == src).astype(f32)
    D = (jrow == dst).astype(f32)
    deg = jnp.sum(D, axis=1, keepdims=True) + 1.0
    dinv = lax.rsqrt(deg)
    dinv_src = jnp.sum(S * dinv, axis=0, keepdims=True)
    dinv_dst = jnp.sum(D * dinv, axis=0, keepdims=True)
    norm = dinv_src * dinv_dst
    At = lax.dot_general(S * norm, D, (((1,), (1,)), ((), ())),
                         preferred_element_type=f32)
    eye = (lax.broadcasted_iota(i32, (J, J), 0)
           == lax.broadcasted_iota(i32, (J, J), 1)).astype(f32)
    At = At + eye * (dinv * dinv)

    p = lax.broadcasted_iota(i32, (64, 1), 0)
    kcol = lax.broadcasted_iota(i32, (64, J), 1)
    R = ((p // F_IN) == kcol).astype(f32)
    q17 = lax.broadcasted_iota(i32, (J, JH), 1)
    jrow2 = lax.broadcasted_iota(i32, (J, JH), 0)
    C = ((q17 // H) == jrow2).astype(f32)
    g1a = jnp.dot(jnp.dot(R, At, preferred_element_type=f32), C,
                  preferred_element_type=f32)
    fcol = lax.broadcasted_iota(i32, (64, F_IN), 1)
    Rf = ((p % F_IN) == fcol).astype(f32)
    qh = lax.broadcasted_iota(i32, (H, JH), 1)
    hrow = lax.broadcasted_iota(i32, (H, JH), 0)
    Ch = ((qh % H) == hrow).astype(f32)
    g1w = jnp.dot(jnp.dot(Rf, w1_ref[...], preferred_element_type=f32), Ch,
                  preferred_element_type=f32)
    bias_row = ((p == JF).astype(f32)
                * jnp.dot(b1_ref[...], Ch, preferred_element_type=f32))
    g1_s[...] = (g1a * g1w + bias_row).astype(jnp.bfloat16)

    p2 = lax.broadcasted_iota(i32, (JH, 1), 0)
    kcol2 = lax.broadcasted_iota(i32, (JH, J), 1)
    R2 = ((p2 // H) == kcol2).astype(f32)
    g2a = jnp.dot(jnp.dot(R2, At, preferred_element_type=f32), C,
                  preferred_element_type=f32)
    hcol2 = lax.broadcasted_iota(i32, (JH, H), 1)
    Rh = ((p2 % H) == hcol2).astype(f32)
    g2w = jnp.dot(jnp.dot(Rh, w2_ref[...], preferred_element_type=f32), Ch,
                  preferred_element_type=f32)
    g2_s[...] = (g2a * g2w).astype(jnp.bfloat16)
    bb2_s[...] = jnp.dot(b2_ref[...], Ch, preferred_element_type=f32)


def _fused_kernel(ei_ref, w1_ref, b1_ref, w2_ref, b2_ref, x_ref, wfc_ref,
                  bfc_ref, out_ref, g1_s, g2_s, bb2_s):
    @pl.when(pl.program_id(0) == 0)
    def _():
        _prep(ei_ref, w1_ref, b1_ref, w2_ref, b2_ref,
              g1_s, g2_s, bb2_s)

    x = x_ref[...]
    g1 = g1_s[0:JFP, :]
    h = jnp.dot(x, g1, preferred_element_type=jnp.float32)
    h = jnp.maximum(h, 0.0)
    h = jnp.dot(h.astype(jnp.bfloat16), g2_s[...],
                preferred_element_type=jnp.float32) + bb2_s[...]
    h = jnp.maximum(h, 0.0)
    out_ref[...] = jnp.dot(h.astype(jnp.bfloat16), wfc_ref[...],
                           preferred_element_type=jnp.float32) + bfc_ref[...]


@jax.jit
def kernel(x, edge_index, W1, b1, W2, b2, Wfc, bfc):
    B = x.shape[0]
    xf = jnp.concatenate(
        [x.reshape(B, JF), jnp.ones((B, 1), x.dtype)],
        axis=1).astype(jnp.bfloat16)
    f32 = jnp.float32
    bf16 = jnp.bfloat16
    out = pl.pallas_call(
        _fused_kernel,
        grid=(B // BT,),
        in_specs=[
            pl.BlockSpec((2, E), lambda i: (0, 0)),
            pl.BlockSpec((F_IN, H), lambda i: (0, 0)),
            pl.BlockSpec((1, H), lambda i: (0, 0)),
            pl.BlockSpec((H, H), lambda i: (0, 0)),
            pl.BlockSpec((1, H), lambda i: (0, 0)),
            pl.BlockSpec((BT, JFP), lambda i: (i, 0)),
            pl.BlockSpec((JH, C_OUT), lambda i: (0, 0)),
            pl.BlockSpec((1, C_OUT), lambda i: (0, 0)),
        ],
        out_specs=pl.BlockSpec((BT, C_OUT), lambda i: (i, 0)),
        out_shape=jax.ShapeDtypeStruct((B, C_OUT), f32),
        scratch_shapes=[
            pltpu.VMEM((64, JH), bf16),
            pltpu.VMEM((JH, JH), bf16),
            pltpu.VMEM((1, JH), f32),
        ],
    )(edge_index, W1, b1.reshape(1, H), W2, b2.reshape(1, H),
      xf, Wfc.astype(bf16), bfc.reshape(1, C_OUT))
    return out


# speedup vs baseline: 1.0456x; 1.0456x over previous
import jax
import jax.numpy as jnp
from jax import lax
from jax.experimental import pallas as pl
from jax.experimental.pallas import tpu as pltpu

J = 17
E = 32
F_IN = 3
H = 64
C_OUT = 16
JF = J * F_IN
JH = J * H
BT = 2048


def _prep(ei_ref, w1_ref, b1_ref, w2_ref, b2_ref,
          g1_s, bb1_s, g2_s, bb2_s):
    f32 = jnp.float32
    i32 = jnp.int32
    ei = ei_ref[...]
    src = ei[0:1, :]
    dst = ei[1:2, :]
    jrow = lax.broadcasted_iota(i32, (J, E), 0)
    S = (jrow == src).astype(f32)
    D = (jrow == dst).astype(f32)
    deg = jnp.sum(D, axis=1, keepdims=True) + 1.0
    dinv = lax.rsqrt(deg)
    dinv_src = jnp.sum(S * dinv, axis=0, keepdims=True)
    dinv_dst = jnp.sum(D * dinv, axis=0, keepdims=True)
    norm = dinv_src * dinv_dst
    At = lax.dot_general(S * norm, D, (((1,), (1,)), ((), ())),
                         preferred_element_type=f32)
    eye = (lax.broadcasted_iota(i32, (J, J), 0)
           == lax.broadcasted_iota(i32, (J, J), 1)).astype(f32)
    At = At + eye * (dinv * dinv)

    p = lax.broadcasted_iota(i32, (64, 1), 0)
    kcol = lax.broadcasted_iota(i32, (64, J), 1)
    R = ((p // F_IN) == kcol).astype(f32)
    q17 = lax.broadcasted_iota(i32, (J, JH), 1)
    jrow2 = lax.broadcasted_iota(i32, (J, JH), 0)
    C = ((q17 // H) == jrow2).astype(f32)
    g1a = jnp.dot(jnp.dot(R, At, preferred_element_type=f32), C,
                  preferred_element_type=f32)
    fcol = lax.broadcasted_iota(i32, (64, F_IN), 1)
    Rf = ((p % F_IN) == fcol).astype(f32)
    qh = lax.broadcasted_iota(i32, (H, JH), 1)
    hrow = lax.broadcasted_iota(i32, (H, JH), 0)
    Ch = ((qh % H) == hrow).astype(f32)
    g1w = jnp.dot(jnp.dot(Rf, w1_ref[...], preferred_element_type=f32), Ch,
                  preferred_element_type=f32)
    g1_s[...] = (g1a * g1w).astype(jnp.bfloat16)
    bb1_s[...] = jnp.dot(b1_ref[...], Ch, preferred_element_type=f32)

    p2 = lax.broadcasted_iota(i32, (JH, 1), 0)
    kcol2 = lax.broadcasted_iota(i32, (JH, J), 1)
    R2 = ((p2 // H) == kcol2).astype(f32)
    g2a = jnp.dot(jnp.dot(R2, At, preferred_element_type=f32), C,
                  preferred_element_type=f32)
    hcol2 = lax.broadcasted_iota(i32, (JH, H), 1)
    Rh = ((p2 % H) == hcol2).astype(f32)
    g2w = jnp.dot(jnp.dot(Rh, w2_ref[...], preferred_element_type=f32), Ch,
                  preferred_element_type=f32)
    g2_s[...] = (g2a * g2w).astype(jnp.bfloat16)
    bb2_s[...] = jnp.dot(b2_ref[...], Ch, preferred_element_type=f32)


def _fused_kernel(ei_ref, w1_ref, b1_ref, w2_ref, b2_ref, x_ref, wfc_ref,
                  bfc_ref, out_ref, g1_s, bb1_s, g2_s, bb2_s):
    @pl.when(pl.program_id(0) == 0)
    def _():
        _prep(ei_ref, w1_ref, b1_ref, w2_ref, b2_ref,
              g1_s, bb1_s, g2_s, bb2_s)

    x = x_ref[...]
    g1 = g1_s[0:JF, :]
    h = jnp.dot(x, g1, preferred_element_type=jnp.float32) + bb1_s[...]
    h = jnp.maximum(h, 0.0)
    h = jnp.dot(h.astype(jnp.bfloat16), g2_s[...],
                preferred_element_type=jnp.float32) + bb2_s[...]
    h = jnp.maximum(h, 0.0)
    out_ref[...] = jnp.dot(h.astype(jnp.bfloat16), wfc_ref[...],
                           preferred_element_type=jnp.float32) + bfc_ref[...]


@jax.jit
def kernel(x, edge_index, W1, b1, W2, b2, Wfc, bfc):
    B = x.shape[0]
    xf = x.reshape(B, JF).astype(jnp.bfloat16)
    f32 = jnp.float32
    bf16 = jnp.bfloat16
    out = pl.pallas_call(
        _fused_kernel,
        grid=(B // BT,),
        in_specs=[
            pl.BlockSpec((2, E), lambda i: (0, 0)),
            pl.BlockSpec((F_IN, H), lambda i: (0, 0)),
            pl.BlockSpec((1, H), lambda i: (0, 0)),
            pl.BlockSpec((H, H), lambda i: (0, 0)),
            pl.BlockSpec((1, H), lambda i: (0, 0)),
            pl.BlockSpec((BT, JF), lambda i: (i, 0)),
            pl.BlockSpec((JH, C_OUT), lambda i: (0, 0)),
            pl.BlockSpec((1, C_OUT), lambda i: (0, 0)),
        ],
        out_specs=pl.BlockSpec((BT, C_OUT), lambda i: (i, 0)),
        out_shape=jax.ShapeDtypeStruct((B, C_OUT), f32),
        scratch_shapes=[
            pltpu.VMEM((64, JH), bf16),
            pltpu.VMEM((1, JH), f32),
            pltpu.VMEM((JH, JH), bf16),
            pltpu.VMEM((1, JH), f32),
        ],
    )(edge_index, W1, b1.reshape(1, H), W2, b2.reshape(1, H),
      xf, Wfc.astype(bf16), bfc.reshape(1, C_OUT))
    return out
